# Initial kernel scaffold; baseline (speedup 1.0000x reference)
import jax
import jax.numpy as jnp
from jax.experimental import pallas as pl


def kernel(x):
    raise NotImplementedError("write your pallas kernel here")


# speedup vs baseline: 15.8326x; 15.8326x over previous
import jax
import jax.numpy as jnp
from jax import lax
from jax.experimental import pallas as pl

_K = 18
_R = 256


def _knn_tc_body(q_ref, xb_ref, vals_ref, idx_ref):
    q = q_ref[0]
    xb = xb_ref[0]
    sq_q = jnp.sum(q * q, axis=-1)
    sq_x = jnp.sum(xb * xb, axis=-1)
    gram = lax.dot_general(q, xb, (((1,), (1,)), ((), ())),
                           preferred_element_type=jnp.float32)
    d2 = sq_q[:, None] + sq_x[None, :] - 2.0 * gram
    d = jnp.sqrt(jnp.maximum(d2, 0.0))

    iota = lax.broadcasted_iota(jnp.int32, d.shape, 1)
    cur = d
    vs, js = [], []
    for _ in range(_K):
        m = jnp.min(cur, axis=1)
        hit = cur == m[:, None]
        am = jnp.min(jnp.where(hit, iota, jnp.int32(2**30)), axis=1)
        vs.append(m)
        js.append(am)
        cur = jnp.where(iota == am[:, None], jnp.float32(jnp.inf), cur)
    vals_ref[0] = jnp.stack(vs, axis=1)
    idx_ref[0] = jnp.stack(js, axis=1)


def kernel(x):
    B, N, D = x.shape
    grid = (B, N // _R)
    vals, idx = pl.pallas_call(
        _knn_tc_body,
        grid=grid,
        in_specs=[pl.BlockSpec((1, _R, D), lambda b, i: (b, i, 0)),
                  pl.BlockSpec((1, N, D), lambda b, i: (b, 0, 0))],
        out_specs=[pl.BlockSpec((1, _R, _K), lambda b, i: (b, i, 0)),
                   pl.BlockSpec((1, _R, _K), lambda b, i: (b, i, 0))],
        out_shape=[jax.ShapeDtypeStruct((B, N, _K), jnp.float32),
                   jax.ShapeDtypeStruct((B, N, _K), jnp.int32)],
    )(x, x)
    return (vals[:, :, 1:], idx[:, :, 1:], x)
